# baseline (speedup 1.0000x reference)
import functools

import jax
import jax.numpy as jnp
from jax import lax
from jax.experimental import pallas as pl
from jax.experimental.pallas import tpu as pltpu
from jax.experimental.pallas import tpu_sc as plsc

B, L, D = 4096, 50, 64
NT = 100000

NC, NS = 2, 16
NW = NC * NS

HB = 1024
RV = HB * L
RV_PER_W = RV // NW
CH = 80
N_CH = RV_PER_W // CH
UB_PER_W = HB // NW

BT = 128
GB = HB // BT

RT = 4096
GT = (NT + RT - 1) // RT


def _pre_body(vt_ref, ut_ref, w1a_ref, a1b_ref, ab1_ref, out_ref):
    f32 = jnp.float32
    bf16 = jnp.bfloat16
    dn = (((0,), (0,)), ((), ()))
    x1 = lax.dot_general(vt_ref[...].astype(bf16), w1a_ref[...], dn,
                         preferred_element_type=f32)
    ut = lax.dot_general(ut_ref[...].astype(bf16), a1b_ref[...], dn,
                         preferred_element_type=f32) + ab1_ref[...]
    out_ref[...] = jnp.concatenate([x1, ut], axis=1)


def _pre_table(v2e_t, u2e_t, w1a_bf, a1b_bf, ab1):
    return pl.pallas_call(
        _pre_body,
        grid=(GT,),
        in_specs=[
            pl.BlockSpec((D, RT), lambda i: (0, i)),
            pl.BlockSpec((D, RT), lambda i: (0, i)),
            pl.BlockSpec((D, D), lambda i: (0, 0)),
            pl.BlockSpec((D, D), lambda i: (0, 0)),
            pl.BlockSpec((1, D), lambda i: (0, 0)),
        ],
        out_specs=pl.BlockSpec((RT, 2 * D), lambda i: (i, 0)),
        out_shape=jax.ShapeDtypeStruct((NT, 2 * D), jnp.float32),
    )(v2e_t, u2e_t, w1a_bf, a1b_bf, ab1)


def _sc_gather(ct, vidx, nidx):
    mesh = plsc.VectorSubcoreMesh(core_axis_name="c", subcore_axis_name="s")

    @functools.partial(
        pl.kernel,
        mesh=mesh,
        out_type=(
            jax.ShapeDtypeStruct((RV, 2 * D), jnp.float32),
            jax.ShapeDtypeStruct((HB, 2 * D), jnp.float32),
        ),
        scratch_types=[
            pltpu.VMEM((RV_PER_W,), jnp.int32),
            pltpu.VMEM((UB_PER_W,), jnp.int32),
            pltpu.VMEM((CH, 2 * D), jnp.float32),
            pltpu.VMEM((CH, 2 * D), jnp.float32),
            pltpu.VMEM((UB_PER_W, 2 * D), jnp.float32),
            pltpu.SemaphoreType.DMA,
            pltpu.SemaphoreType.DMA,
            pltpu.SemaphoreType.DMA,
            pltpu.SemaphoreType.DMA,
            pltpu.SemaphoreType.DMA,
        ],
    )
    def k(ct_hbm, vidx_hbm, nidx_hbm, euv_out, uv_out,
          vidx_v, nidx_v, rows0, rows1, urows,
          sg0, sg1, sw0, sw1, su):
        wid = lax.axis_index("s") * NC + lax.axis_index("c")
        base = wid * RV_PER_W
        pltpu.sync_copy(vidx_hbm.at[pl.ds(base, RV_PER_W)], vidx_v)

        ubase = wid * UB_PER_W
        pltpu.sync_copy(nidx_hbm.at[pl.ds(ubase, UB_PER_W)], nidx_v)
        pltpu.async_copy(ct_hbm.at[nidx_v], urows, su)

        def start_g(c, rows, sem):
            pltpu.async_copy(
                ct_hbm.at[vidx_v.at[pl.ds(c * CH, CH)]], rows, sem)

        def wait_g(c, rows, sem):
            pltpu.make_async_copy(
                ct_hbm.at[vidx_v.at[pl.ds(c * CH, CH)]], rows, sem).wait()

        def start_w(c, rows, sem):
            pltpu.async_copy(
                rows, euv_out.at[pl.ds(base + c * CH, CH)], sem)

        def wait_w(c, rows, sem):
            pltpu.make_async_copy(
                rows, euv_out.at[pl.ds(base + c * CH, CH)], sem).wait()

        start_g(0, rows0, sg0)

        def body(i, carry):
            c0 = 2 * i
            c1 = c0 + 1
            wait_g(c0, rows0, sg0)

            @pl.when(i > 0)
            def _():
                wait_w(c1 - 2, rows1, sw1)

            start_g(c1, rows1, sg1)
            start_w(c0, rows0, sw0)

            @pl.when(i < N_CH // 2 - 1)
            def _():
                wait_w(c0, rows0, sw0)
                start_g(c0 + 2, rows0, sg0)

            wait_g(c1, rows1, sg1)
            start_w(c1, rows1, sw1)
            return carry

        lax.fori_loop(0, N_CH // 2, body, 0)
        if N_CH % 2:
            wait_w(N_CH - 3, rows0, sw0)
            start_g(N_CH - 1, rows0, sg0)
            wait_g(N_CH - 1, rows0, sg0)
            start_w(N_CH - 1, rows0, sw0)
            wait_w(N_CH - 1, rows0, sw0)
            wait_w(N_CH - 2, rows1, sw1)
        else:
            wait_w(N_CH - 2, rows0, sw0)
            wait_w(N_CH - 1, rows1, sw1)

        pltpu.make_async_copy(ct_hbm.at[nidx_v], urows, su).wait()
        pltpu.sync_copy(urows, uv_out.at[pl.ds(ubase, UB_PER_W)])

    return k(ct, vidx, nidx)


def _tc_body(euv_ref, uv_ref, roh_ref, r2e_ref,
             w1b_ref, b1_ref, w2_ref, b2_ref,
             a1a_ref, a2_ref, ab2_ref,
             a3_ref, ab3_ref, out_ref):
    f32 = jnp.float32
    bf16 = jnp.bfloat16
    mm = lambda a, b: jnp.dot(a, b, preferred_element_type=f32).astype(bf16)
    zero = jnp.array(0.0, bf16)
    blk = euv_ref[...]
    x1 = blk[:, :, :D].reshape(L * BT, D).astype(bf16)
    rt = (jnp.dot(r2e_ref[...], w1b_ref[...],
                  preferred_element_type=f32) + b1_ref[...]).astype(bf16)
    oh = roh_ref[...].reshape(L * BT, 8).astype(bf16)
    x = jnp.maximum(x1 + mm(oh, rt), zero)
    o = jnp.maximum(mm(x, w2_ref[...]) + b2_ref[...], zero)
    u_term = uv_ref[...][:, D:].astype(bf16)
    u_rep = jnp.broadcast_to(u_term[None], (L, BT, D)).reshape(L * BT, D)
    a1 = jnp.maximum(mm(o, a1a_ref[...]) + u_rep, zero)
    a2 = jnp.maximum(mm(a1, a2_ref[...]) + ab2_ref[...], zero)
    logits = jnp.sum((a2 * a3_ref[...]).astype(f32), axis=1,
                     keepdims=True) + ab3_ref[...]
    l3 = logits.reshape(L, BT, 1)
    m = jnp.max(l3, axis=0)
    e3 = jnp.exp(l3 - m[None])
    s = jnp.sum(e3, axis=0)
    o3 = o.reshape(L, BT, D).astype(f32)
    out_ref[...] = jnp.sum(o3 * e3, axis=0) / s


def _tc_dense(euv3, uv, roh, r2e_pad, w1b_t, b1r, w2_t, b2r,
              a1a_t, a2_t, ab2r, a3r, ab3r):
    full = lambda shape: pl.BlockSpec(shape, lambda i: tuple(0 for _ in shape))
    return pl.pallas_call(
        _tc_body,
        grid=(GB,),
        in_specs=[
            pl.BlockSpec((L, BT, 2 * D), lambda i: (0, i, 0)),
            pl.BlockSpec((BT, 2 * D), lambda i: (i, 0)),
            pl.BlockSpec((L, BT, 8), lambda i: (0, i, 0)),
            full((8, D)),
            full((D, D)), full((1, D)),
            full((D, D)), full((1, D)),
            full((D, D)),
            full((D, D)), full((1, D)),
            full((1, D)), full((1, 1)),
        ],
        out_specs=pl.BlockSpec((BT, D), lambda i: (i, 0)),
        out_shape=jax.ShapeDtypeStruct((HB, D), jnp.float32),
    )(euv3, uv, roh, r2e_pad, w1b_t, b1r, w2_t, b2r, a1a_t, a2_t, ab2r,
      a3r, ab3r)


def kernel(nodes, history_uv, history_r, u2e, v2e, r2e,
           w_r1_w, w_r1_b, w_r2_w, w_r2_b,
           att1_w, att1_b, att2_w, att2_b, att3_w, att3_b):
    f32 = jnp.float32
    bf16 = jnp.bfloat16

    ct = _pre_table(v2e.T, u2e.T,
                    w_r1_w[:, :D].T.astype(bf16),
                    att1_w[:, D:].T.astype(bf16),
                    att1_b.reshape(1, D))

    r2e_pad = jnp.pad(r2e, ((0, 3), (0, 0)))
    outs = []
    for h in range(B // HB):
        sl = slice(h * HB, (h + 1) * HB)
        vidx = history_uv[sl].T.reshape(-1).astype(jnp.int32)
        nidx = nodes[sl].astype(jnp.int32)
        roh = jax.nn.one_hot(history_r[sl].T, 8, dtype=jnp.int8)
        euv_flat, uv = _sc_gather(ct, vidx, nidx)
        outs.append(_tc_dense(
            euv_flat.reshape(L, HB, 2 * D), uv, roh, r2e_pad,
            w_r1_w[:, D:].T, w_r1_b.reshape(1, D),
            w_r2_w.T.astype(bf16), w_r2_b.reshape(1, D).astype(bf16),
            att1_w[:, :D].T.astype(bf16),
            att2_w.T.astype(bf16), att2_b.reshape(1, D).astype(bf16),
            att3_w.astype(bf16), att3_b.reshape(1, 1),
        ))
    return jnp.concatenate(outs, axis=0)


# speedup vs baseline: 1.0516x; 1.0516x over previous
import functools

import jax
import jax.numpy as jnp
from jax import lax
from jax.experimental import pallas as pl
from jax.experimental.pallas import tpu as pltpu
from jax.experimental.pallas import tpu_sc as plsc

B, L, D = 4096, 50, 64
NT = 100000

NC, NS = 2, 16
NW = NC * NS

HB = 2048
RV = HB * L
RV_PER_W = RV // NW
CH = 128
N_CH = RV_PER_W // CH
UB_PER_W = HB // NW

BT = 128
GB = HB // BT

RT = 8192
GT = (NT + RT - 1) // RT


def _pre_body(vt_ref, ut_ref, w1a_ref, a1b_ref, ab1_ref, out_ref):
    f32 = jnp.float32
    bf16 = jnp.bfloat16
    dn = (((0,), (0,)), ((), ()))
    x1 = lax.dot_general(vt_ref[...].astype(bf16), w1a_ref[...], dn,
                         preferred_element_type=f32)
    ut = lax.dot_general(ut_ref[...].astype(bf16), a1b_ref[...], dn,
                         preferred_element_type=f32) + ab1_ref[...]
    out_ref[...] = jnp.concatenate([x1, ut], axis=1)


def _pre_table(v2e_t, u2e_t, w1a_bf, a1b_bf, ab1):
    return pl.pallas_call(
        _pre_body,
        grid=(GT,),
        in_specs=[
            pl.BlockSpec((D, RT), lambda i: (0, i)),
            pl.BlockSpec((D, RT), lambda i: (0, i)),
            pl.BlockSpec((D, D), lambda i: (0, 0)),
            pl.BlockSpec((D, D), lambda i: (0, 0)),
            pl.BlockSpec((1, D), lambda i: (0, 0)),
        ],
        out_specs=pl.BlockSpec((RT, 2 * D), lambda i: (i, 0)),
        out_shape=jax.ShapeDtypeStruct((NT, 2 * D), jnp.float32),
    )(v2e_t, u2e_t, w1a_bf, a1b_bf, ab1)


def _sc_gather(ct, vidx, nidx):
    mesh = plsc.VectorSubcoreMesh(core_axis_name="c", subcore_axis_name="s")

    @functools.partial(
        pl.kernel,
        mesh=mesh,
        out_type=(
            jax.ShapeDtypeStruct((RV, 2 * D), jnp.float32),
            jax.ShapeDtypeStruct((HB, 2 * D), jnp.float32),
        ),
        scratch_types=[
            pltpu.VMEM((RV_PER_W,), jnp.int32),
            pltpu.VMEM((UB_PER_W,), jnp.int32),
            pltpu.VMEM((CH, 2 * D), jnp.float32),
            pltpu.VMEM((CH, 2 * D), jnp.float32),
            pltpu.VMEM((UB_PER_W, 2 * D), jnp.float32),
            pltpu.SemaphoreType.DMA,
            pltpu.SemaphoreType.DMA,
            pltpu.SemaphoreType.DMA,
            pltpu.SemaphoreType.DMA,
            pltpu.SemaphoreType.DMA,
        ],
    )
    def k(ct_hbm, vidx_hbm, nidx_hbm, euv_out, uv_out,
          vidx_v, nidx_v, rows0, rows1, urows,
          sg0, sg1, sw0, sw1, su):
        wid = lax.axis_index("s") * NC + lax.axis_index("c")
        base = wid * RV_PER_W
        pltpu.sync_copy(vidx_hbm.at[pl.ds(base, RV_PER_W)], vidx_v)

        ubase = wid * UB_PER_W
        pltpu.sync_copy(nidx_hbm.at[pl.ds(ubase, UB_PER_W)], nidx_v)
        pltpu.async_copy(ct_hbm.at[nidx_v], urows, su)

        def start_g(c, rows, sem):
            pltpu.async_copy(
                ct_hbm.at[vidx_v.at[pl.ds(c * CH, CH)]], rows, sem)

        def wait_g(c, rows, sem):
            pltpu.make_async_copy(
                ct_hbm.at[vidx_v.at[pl.ds(c * CH, CH)]], rows, sem).wait()

        def start_w(c, rows, sem):
            pltpu.async_copy(
                rows, euv_out.at[pl.ds(base + c * CH, CH)], sem)

        def wait_w(c, rows, sem):
            pltpu.make_async_copy(
                rows, euv_out.at[pl.ds(base + c * CH, CH)], sem).wait()

        start_g(0, rows0, sg0)

        def body(i, carry):
            c0 = 2 * i
            c1 = c0 + 1
            wait_g(c0, rows0, sg0)

            @pl.when(i > 0)
            def _():
                wait_w(c1 - 2, rows1, sw1)

            start_g(c1, rows1, sg1)
            start_w(c0, rows0, sw0)

            @pl.when(i < N_CH // 2 - 1)
            def _():
                wait_w(c0, rows0, sw0)
                start_g(c0 + 2, rows0, sg0)

            wait_g(c1, rows1, sg1)
            start_w(c1, rows1, sw1)
            return carry

        lax.fori_loop(0, N_CH // 2, body, 0)
        if N_CH % 2:
            wait_w(N_CH - 3, rows0, sw0)
            start_g(N_CH - 1, rows0, sg0)
            wait_g(N_CH - 1, rows0, sg0)
            start_w(N_CH - 1, rows0, sw0)
            wait_w(N_CH - 1, rows0, sw0)
            wait_w(N_CH - 2, rows1, sw1)
        else:
            wait_w(N_CH - 2, rows0, sw0)
            wait_w(N_CH - 1, rows1, sw1)

        pltpu.make_async_copy(ct_hbm.at[nidx_v], urows, su).wait()
        pltpu.sync_copy(urows, uv_out.at[pl.ds(ubase, UB_PER_W)])

    return k(ct, vidx, nidx)


def _tc_body(euv_ref, uv_ref, roh_ref, r2e_ref,
             w1b_ref, b1_ref, w2_ref, b2_ref,
             a1a_ref, a2_ref, ab2_ref,
             a3_ref, ab3_ref, out_ref):
    f32 = jnp.float32
    bf16 = jnp.bfloat16
    mm = lambda a, b: jnp.dot(a, b, preferred_element_type=f32).astype(bf16)
    zero = jnp.array(0.0, bf16)
    blk = euv_ref[...]
    x1 = blk[:, :, :D].reshape(L * BT, D).astype(bf16)
    rt = (jnp.dot(r2e_ref[...], w1b_ref[...],
                  preferred_element_type=f32) + b1_ref[...]).astype(bf16)
    oh = roh_ref[...].reshape(L * BT, 8).astype(bf16)
    x = jnp.maximum(x1 + mm(oh, rt), zero)
    o = jnp.maximum(mm(x, w2_ref[...]) + b2_ref[...], zero)
    u_term = uv_ref[...][:, D:].astype(bf16)
    u_rep = jnp.broadcast_to(u_term[None], (L, BT, D)).reshape(L * BT, D)
    a1 = jnp.maximum(mm(o, a1a_ref[...]) + u_rep, zero)
    a2 = jnp.maximum(mm(a1, a2_ref[...]) + ab2_ref[...], zero)
    logits = jnp.sum((a2 * a3_ref[...]).astype(f32), axis=1,
                     keepdims=True) + ab3_ref[...]
    l3 = logits.reshape(L, BT, 1)
    m = jnp.max(l3, axis=0)
    e3 = jnp.exp(l3 - m[None])
    s = jnp.sum(e3, axis=0)
    o3 = o.reshape(L, BT, D).astype(f32)
    out_ref[...] = jnp.sum(o3 * e3, axis=0) / s


def _tc_dense(euv3, uv, roh, r2e_pad, w1b_t, b1r, w2_t, b2r,
              a1a_t, a2_t, ab2r, a3r, ab3r):
    full = lambda shape: pl.BlockSpec(shape, lambda i: tuple(0 for _ in shape))
    return pl.pallas_call(
        _tc_body,
        grid=(GB,),
        in_specs=[
            pl.BlockSpec((L, BT, 2 * D), lambda i: (0, i, 0)),
            pl.BlockSpec((BT, 2 * D), lambda i: (i, 0)),
            pl.BlockSpec((L, BT, 8), lambda i: (0, i, 0)),
            full((8, D)),
            full((D, D)), full((1, D)),
            full((D, D)), full((1, D)),
            full((D, D)),
            full((D, D)), full((1, D)),
            full((1, D)), full((1, 1)),
        ],
        out_specs=pl.BlockSpec((BT, D), lambda i: (i, 0)),
        out_shape=jax.ShapeDtypeStruct((HB, D), jnp.float32),
    )(euv3, uv, roh, r2e_pad, w1b_t, b1r, w2_t, b2r, a1a_t, a2_t, ab2r,
      a3r, ab3r)


def kernel(nodes, history_uv, history_r, u2e, v2e, r2e,
           w_r1_w, w_r1_b, w_r2_w, w_r2_b,
           att1_w, att1_b, att2_w, att2_b, att3_w, att3_b):
    f32 = jnp.float32
    bf16 = jnp.bfloat16

    ct = _pre_table(v2e.T, u2e.T,
                    w_r1_w[:, :D].T.astype(bf16),
                    att1_w[:, D:].T.astype(bf16),
                    att1_b.reshape(1, D))

    r2e_pad = jnp.pad(r2e, ((0, 3), (0, 0)))
    outs = []
    for h in range(B // HB):
        sl = slice(h * HB, (h + 1) * HB)
        vidx = history_uv[sl].T.reshape(-1).astype(jnp.int32)
        nidx = nodes[sl].astype(jnp.int32)
        roh = jax.nn.one_hot(history_r[sl].T, 8, dtype=jnp.int8)
        euv_flat, uv = _sc_gather(ct, vidx, nidx)
        outs.append(_tc_dense(
            euv_flat.reshape(L, HB, 2 * D), uv, roh, r2e_pad,
            w_r1_w[:, D:].T, w_r1_b.reshape(1, D),
            w_r2_w.T.astype(bf16), w_r2_b.reshape(1, D).astype(bf16),
            att1_w[:, :D].T.astype(bf16),
            att2_w.T.astype(bf16), att2_b.reshape(1, D).astype(bf16),
            att3_w.astype(bf16), att3_b.reshape(1, 1),
        ))
    return jnp.concatenate(outs, axis=0)
